# initial kernel scaffold (unmeasured)
import functools

import jax
import jax.numpy as jnp
from jax import lax
from jax.experimental import pallas as pl
from jax.experimental.pallas import tpu as pltpu

N_DEV = 16
B_LOC = 2
SQ = 256
SKV = 256
D_MODEL = 512
HQ_LOC = 4
DH = 64
D_HEADS_LOC = HQ_LOC * DH
BLK = 64


def _body(x_ref, wq_ref, wo_ref, k_ref, v_ref, out_ref,
          wq_g, wo_g, sq_send, sq_recv, so_send, so_recv):
    my = lax.axis_index("i")
    left = lax.rem(my + N_DEV - 1, N_DEV)
    right = lax.rem(my + 1, N_DEV)

    barrier_sem = pltpu.get_barrier_semaphore()
    for nbr in (left, right):
        pl.semaphore_signal(
            barrier_sem, inc=1,
            device_id=(nbr,), device_id_type=pl.DeviceIdType.MESH,
        )
    pl.semaphore_wait(barrier_sem, 2)

    wq_g[pl.ds(my, 1)] = wq_ref[:, :].reshape(1, D_MODEL, D_HEADS_LOC)
    wo_g[pl.ds(my, 1)] = wo_ref[:, :].reshape(1, D_HEADS_LOC, D_MODEL)

    for h in range(N_DEV - 1):
        s = lax.rem(my - h + N_DEV, N_DEV)
        rq = pltpu.make_async_remote_copy(
            src_ref=wq_g.at[s], dst_ref=wq_g.at[s],
            send_sem=sq_send.at[h], recv_sem=sq_recv.at[h],
            device_id=(right,), device_id_type=pl.DeviceIdType.MESH,
        )
        ro = pltpu.make_async_remote_copy(
            src_ref=wo_g.at[s], dst_ref=wo_g.at[s],
            send_sem=so_send.at[h], recv_sem=so_recv.at[h],
            device_id=(right,), device_id_type=pl.DeviceIdType.MESH,
        )
        rq.start()
        ro.start()
        rq.wait()
        ro.wait()

    row_blk = lax.broadcasted_iota(jnp.int32, (SQ, SKV), 0) // BLK
    col_blk = lax.broadcasted_iota(jnp.int32, (SQ, SKV), 1) // BLK
    mask = col_blk <= row_blk

    for b in range(B_LOC):
        xb = x_ref[b]
        acc = jnp.zeros((SQ, D_MODEL), jnp.float32)
        for hblk in range(N_DEV):
            q = jnp.dot(xb, wq_g[hblk],
                        preferred_element_type=jnp.float32)
            ctx_parts = []
            for j in range(HQ_LOC):
                head = HQ_LOC * hblk + j
                qj = q[:, DH * j:DH * (j + 1)]
                kj = k_ref[head, b]
                vj = v_ref[head, b]
                s_ = lax.dot_general(
                    qj, kj, (((1,), (1,)), ((), ())),
                    preferred_element_type=jnp.float32) * 0.125
                s_ = jnp.where(mask, s_, -1e9)
                m = jnp.max(s_, axis=1, keepdims=True)
                w = jnp.exp(s_ - m)
                w = w / jnp.sum(w, axis=1, keepdims=True)
                ctx_parts.append(
                    jnp.dot(w, vj, preferred_element_type=jnp.float32))
            ctx = jnp.concatenate(ctx_parts, axis=1)
            acc = acc + jnp.dot(ctx, wo_g[hblk],
                                preferred_element_type=jnp.float32)
        out_ref[b] = acc

    @functools.partial(pl.run_scoped, second_barrier=pltpu.SemaphoreType.REGULAR)
    def _(second_barrier):
        for nbr in (left, right):
            pl.semaphore_signal(
                second_barrier, inc=1,
                device_id=(nbr,), device_id_type=pl.DeviceIdType.MESH,
            )
        pl.semaphore_wait(second_barrier, 2)


def kernel(x, Wq, K_ext, V_ext, Wo):
    my = lax.axis_index("i")
    k_loc = lax.dynamic_slice_in_dim(K_ext, my * B_LOC, B_LOC, axis=0)
    v_loc = lax.dynamic_slice_in_dim(V_ext, my * B_LOC, B_LOC, axis=0)
    k_t = k_loc.transpose(2, 0, 1, 3)
    v_t = v_loc.transpose(2, 0, 1, 3)

    return pl.pallas_call(
        _body,
        out_shape=jax.ShapeDtypeStruct((B_LOC, SQ, D_MODEL), jnp.float32),
        in_specs=[pl.BlockSpec(memory_space=pltpu.VMEM)] * 5,
        out_specs=pl.BlockSpec(memory_space=pltpu.VMEM),
        scratch_shapes=[
            pltpu.VMEM((N_DEV, D_MODEL, D_HEADS_LOC), jnp.float32),
            pltpu.VMEM((N_DEV, D_HEADS_LOC, D_MODEL), jnp.float32),
            pltpu.SemaphoreType.DMA((N_DEV - 1,)),
            pltpu.SemaphoreType.DMA((N_DEV - 1,)),
            pltpu.SemaphoreType.DMA((N_DEV - 1,)),
            pltpu.SemaphoreType.DMA((N_DEV - 1,)),
        ],
        compiler_params=pltpu.CompilerParams(collective_id=0),
    )(x, Wq, Wo, k_t, v_t)


# baseline (device time: 277778 ns/iter reference)
import functools

import jax
import jax.numpy as jnp
from jax import lax
from jax.experimental import pallas as pl
from jax.experimental.pallas import tpu as pltpu

N_DEV = 16
B_LOC = 2
SQ = 256
SKV = 256
D_MODEL = 512
HQ_LOC = 4
DH = 64
D_HEADS_LOC = HQ_LOC * DH
BLK = 64


def _body(x_ref, wq_ref, wo_ref, k_ref, v_ref, out_ref,
          wq_g, wo_g, sq_send, sq_recv, so_send, so_recv):
    my = lax.axis_index("i")
    left = lax.rem(my + N_DEV - 1, N_DEV)
    right = lax.rem(my + 1, N_DEV)

    barrier_sem = pltpu.get_barrier_semaphore()
    for nbr in (left, right):
        pl.semaphore_signal(
            barrier_sem, inc=1,
            device_id=(nbr,), device_id_type=pl.DeviceIdType.MESH,
        )
    pl.semaphore_wait(barrier_sem, 2)

    wq_g[pl.ds(my, 1)] = wq_ref[:, :].reshape(1, D_MODEL, D_HEADS_LOC)
    wo_g[pl.ds(my, 1)] = wo_ref[:, :].reshape(1, D_HEADS_LOC, D_MODEL)

    for h in range(N_DEV - 1):
        s = lax.rem(my - h + N_DEV, N_DEV)
        rq = pltpu.make_async_remote_copy(
            src_ref=wq_g.at[s], dst_ref=wq_g.at[s],
            send_sem=sq_send.at[h], recv_sem=sq_recv.at[h],
            device_id=(right,), device_id_type=pl.DeviceIdType.MESH,
        )
        ro = pltpu.make_async_remote_copy(
            src_ref=wo_g.at[s], dst_ref=wo_g.at[s],
            send_sem=so_send.at[h], recv_sem=so_recv.at[h],
            device_id=(right,), device_id_type=pl.DeviceIdType.MESH,
        )
        rq.start()
        ro.start()
        rq.wait()
        ro.wait()

    row_blk = lax.broadcasted_iota(jnp.int32, (SQ, SKV), 0) // BLK
    col_blk = lax.broadcasted_iota(jnp.int32, (SQ, SKV), 1) // BLK
    mask = col_blk <= row_blk

    for b in range(B_LOC):
        xb = x_ref[b]
        acc = jnp.zeros((SQ, D_MODEL), jnp.float32)
        for hblk in range(N_DEV):
            q = jnp.dot(xb, wq_g[hblk],
                        preferred_element_type=jnp.float32)
            ctx_parts = []
            for j in range(HQ_LOC):
                head = HQ_LOC * hblk + j
                qj = q[:, DH * j:DH * (j + 1)]
                kj = k_ref[head, b]
                vj = v_ref[head, b]
                s_ = lax.dot_general(
                    qj, kj, (((1,), (1,)), ((), ())),
                    preferred_element_type=jnp.float32) * 0.125
                s_ = jnp.where(mask, s_, -1e9)
                m = jnp.max(s_, axis=1, keepdims=True)
                w = jnp.exp(s_ - m)
                w = w / jnp.sum(w, axis=1, keepdims=True)
                ctx_parts.append(
                    jnp.dot(w, vj, preferred_element_type=jnp.float32))
            ctx = jnp.concatenate(ctx_parts, axis=1)
            acc = acc + jnp.dot(ctx, wo_g[hblk],
                                preferred_element_type=jnp.float32)
        out_ref[b] = acc

    @functools.partial(pl.run_scoped, second_barrier=pltpu.SemaphoreType.REGULAR)
    def _(second_barrier):
        for nbr in (left, right):
            pl.semaphore_signal(
                second_barrier, inc=1,
                device_id=(nbr,), device_id_type=pl.DeviceIdType.MESH,
            )
        pl.semaphore_wait(second_barrier, 2)


def kernel(x, Wq, K_ext, V_ext, Wo):
    my = lax.axis_index("i")
    k_loc = lax.dynamic_slice_in_dim(K_ext, my * B_LOC, B_LOC, axis=0)
    v_loc = lax.dynamic_slice_in_dim(V_ext, my * B_LOC, B_LOC, axis=0)
    k_t = k_loc.transpose(2, 0, 1, 3)
    v_t = v_loc.transpose(2, 0, 1, 3)

    return pl.pallas_call(
        _body,
        out_shape=jax.ShapeDtypeStruct((B_LOC, SQ, D_MODEL), jnp.float32),
        in_specs=[pl.BlockSpec(memory_space=pltpu.VMEM)] * 5,
        out_specs=pl.BlockSpec(memory_space=pltpu.VMEM),
        scratch_shapes=[
            pltpu.VMEM((N_DEV, D_MODEL, D_HEADS_LOC), jnp.float32),
            pltpu.VMEM((N_DEV, D_HEADS_LOC, D_MODEL), jnp.float32),
            pltpu.SemaphoreType.DMA((N_DEV - 1,)),
            pltpu.SemaphoreType.DMA((N_DEV - 1,)),
            pltpu.SemaphoreType.DMA((N_DEV - 1,)),
            pltpu.SemaphoreType.DMA((N_DEV - 1,)),
        ],
        compiler_params=pltpu.CompilerParams(
            collective_id=0, vmem_limit_bytes=100 * 1024 * 1024
        ),
    )(x, Wq, Wo, k_t, v_t)


# device time: 160587 ns/iter; 1.7298x vs baseline; 1.7298x over previous
import functools

import jax
import jax.numpy as jnp
from jax import lax
from jax.experimental import pallas as pl
from jax.experimental.pallas import tpu as pltpu

N_DEV = 16
B_LOC = 2
SQ = 256
SKV = 256
D_MODEL = 512
HQ_LOC = 4
DH = 64
D_HEADS_LOC = HQ_LOC * DH
BLK = 64
R_HOPS = 8
L_HOPS = 7


def _body(x_ref, wq_ref, wo_ref, k_ref, v_ref, out_ref,
          wq_g, wo_g,
          rq_send, rq_recv, ro_send, ro_recv,
          lq_send, lq_recv, lo_send, lo_recv):
    my = lax.axis_index("i")
    left = lax.rem(my + N_DEV - 1, N_DEV)
    right = lax.rem(my + 1, N_DEV)

    barrier_sem = pltpu.get_barrier_semaphore()
    for nbr in (left, right):
        pl.semaphore_signal(
            barrier_sem, inc=1,
            device_id=(nbr,), device_id_type=pl.DeviceIdType.MESH,
        )
    pl.semaphore_wait(barrier_sem, 2)

    wq_g[0] = wq_ref[:, :]
    wo_g[0] = wo_ref[:, :]

    mask = (lax.broadcasted_iota(jnp.int32, (SQ, SKV), 1) // BLK) <= (
        lax.broadcasted_iota(jnp.int32, (SQ, SKV), 0) // BLK)

    xb = [x_ref[b] for b in range(B_LOC)]
    acc = [jnp.zeros((SQ, D_MODEL), jnp.float32) for _ in range(B_LOC)]

    def contrib(slot):
        o = lax.rem(my + slot, N_DEV)
        wq_blk = wq_g[slot]
        wo_blk = wo_g[slot]
        for b in range(B_LOC):
            q = jnp.dot(xb[b], wq_blk,
                        preferred_element_type=jnp.float32)
            ctx_parts = []
            for j in range(HQ_LOC):
                qj = q[:, DH * j:DH * (j + 1)]
                kj = k_ref[pl.ds(HQ_LOC * o + j, 1), b].reshape(SKV, DH)
                vj = v_ref[pl.ds(HQ_LOC * o + j, 1), b].reshape(SKV, DH)
                s_ = lax.dot_general(
                    qj, kj, (((1,), (1,)), ((), ())),
                    preferred_element_type=jnp.float32) * 0.125
                s_ = jnp.where(mask, s_, -1e9)
                m = jnp.max(s_, axis=1, keepdims=True)
                w = jnp.exp(s_ - m)
                w = w / jnp.sum(w, axis=1, keepdims=True)
                ctx_parts.append(
                    jnp.dot(w, vj, preferred_element_type=jnp.float32))
            ctx = jnp.concatenate(ctx_parts, axis=1)
            acc[b] = acc[b] + jnp.dot(
                ctx, wo_blk, preferred_element_type=jnp.float32)

    def make_hop(h, rightward):
        if rightward:
            src, dst, tgt = (0 if h == 0 else N_DEV - h), N_DEV - 1 - h, right
            sems = (rq_send, rq_recv, ro_send, ro_recv)
        else:
            src, dst, tgt = (0 if h == 0 else h), h + 1, left
            sems = (lq_send, lq_recv, lo_send, lo_recv)
        rq = pltpu.make_async_remote_copy(
            src_ref=wq_g.at[src], dst_ref=wq_g.at[dst],
            send_sem=sems[0].at[h], recv_sem=sems[1].at[h],
            device_id=(tgt,), device_id_type=pl.DeviceIdType.MESH,
        )
        ro = pltpu.make_async_remote_copy(
            src_ref=wo_g.at[src], dst_ref=wo_g.at[dst],
            send_sem=sems[2].at[h], recv_sem=sems[3].at[h],
            device_id=(tgt,), device_id_type=pl.DeviceIdType.MESH,
        )
        return rq, ro

    hops_r = [make_hop(h, True) for h in range(R_HOPS)]
    hops_l = [make_hop(h, False) for h in range(L_HOPS)]

    for r in hops_r[0] + hops_l[0]:
        r.start()
    contrib(0)

    for h in range(R_HOPS):
        for r in hops_r[h]:
            r.wait_recv()
        if h < L_HOPS:
            for r in hops_l[h]:
                r.wait_recv()
        if h + 1 < R_HOPS:
            for r in hops_r[h + 1]:
                r.start()
        if h + 1 < L_HOPS:
            for r in hops_l[h + 1]:
                r.start()
        for r in hops_r[h]:
            r.wait_send()
        if h < L_HOPS:
            for r in hops_l[h]:
                r.wait_send()
        contrib(N_DEV - 1 - h)
        if h < L_HOPS:
            contrib(h + 1)

    for b in range(B_LOC):
        out_ref[b] = acc[b]

    @functools.partial(pl.run_scoped, second_barrier=pltpu.SemaphoreType.REGULAR)
    def _(second_barrier):
        for nbr in (left, right):
            pl.semaphore_signal(
                second_barrier, inc=1,
                device_id=(nbr,), device_id_type=pl.DeviceIdType.MESH,
            )
        pl.semaphore_wait(second_barrier, 2)


def kernel(x, Wq, K_ext, V_ext, Wo):
    my = lax.axis_index("i")
    k_loc = lax.dynamic_slice_in_dim(K_ext, my * B_LOC, B_LOC, axis=0)
    v_loc = lax.dynamic_slice_in_dim(V_ext, my * B_LOC, B_LOC, axis=0)
    k_t = k_loc.transpose(2, 0, 1, 3)
    v_t = v_loc.transpose(2, 0, 1, 3)

    return pl.pallas_call(
        _body,
        out_shape=jax.ShapeDtypeStruct((B_LOC, SQ, D_MODEL), jnp.float32),
        in_specs=[pl.BlockSpec(memory_space=pltpu.VMEM)] * 5,
        out_specs=pl.BlockSpec(memory_space=pltpu.VMEM),
        scratch_shapes=[
            pltpu.VMEM((N_DEV, D_MODEL, D_HEADS_LOC), jnp.float32),
            pltpu.VMEM((N_DEV, D_HEADS_LOC, D_MODEL), jnp.float32),
            pltpu.SemaphoreType.DMA((R_HOPS,)),
            pltpu.SemaphoreType.DMA((R_HOPS,)),
            pltpu.SemaphoreType.DMA((R_HOPS,)),
            pltpu.SemaphoreType.DMA((R_HOPS,)),
            pltpu.SemaphoreType.DMA((L_HOPS,)),
            pltpu.SemaphoreType.DMA((L_HOPS,)),
            pltpu.SemaphoreType.DMA((L_HOPS,)),
            pltpu.SemaphoreType.DMA((L_HOPS,)),
        ],
        compiler_params=pltpu.CompilerParams(
            collective_id=0, vmem_limit_bytes=100 * 1024 * 1024
        ),
    )(x, Wq, Wo, k_t, v_t)


# device time: 108446 ns/iter; 2.5614x vs baseline; 1.4808x over previous
import functools

import jax
import jax.numpy as jnp
from jax import lax
from jax.experimental import pallas as pl
from jax.experimental.pallas import tpu as pltpu

N_DEV = 16
B_LOC = 2
SQ = 256
SKV = 256
D_MODEL = 512
HQ_LOC = 4
DH = 64
D_HEADS_LOC = HQ_LOC * DH
BLK = 64
R_HOPS = 8
L_HOPS = 7


def _body(x_ref, wq_ref, wo_ref, k_ref, v_ref, out_ref,
          wq_g, wo_g,
          rq_send, rq_recv, ro_send, ro_recv,
          lq_send, lq_recv, lo_send, lo_recv):
    my = lax.axis_index("i")
    left = lax.rem(my + N_DEV - 1, N_DEV)
    right = lax.rem(my + 1, N_DEV)

    barrier_sem = pltpu.get_barrier_semaphore()
    for nbr in (left, right):
        pl.semaphore_signal(
            barrier_sem, inc=1,
            device_id=(nbr,), device_id_type=pl.DeviceIdType.MESH,
        )
    pl.semaphore_wait(barrier_sem, 2)

    wq_g[0] = wq_ref[:, :]
    wo_g[0] = wo_ref[:, :]

    mask = (lax.broadcasted_iota(jnp.int32, (SQ, SKV), 1) // BLK) <= (
        lax.broadcasted_iota(jnp.int32, (SQ, SKV), 0) // BLK)

    xb = [x_ref[b] for b in range(B_LOC)]
    acc = [jnp.zeros((SQ, D_MODEL), jnp.float32) for _ in range(B_LOC)]

    def contrib(slot):
        o = lax.rem(my + slot, N_DEV)
        wq_blk = wq_g[slot]
        wo_blk = wo_g[slot]
        for b in range(B_LOC):
            q = jnp.dot(xb[b], wq_blk,
                        preferred_element_type=jnp.float32)
            ctx_parts = []
            for j in range(HQ_LOC):
                qj = q[:, DH * j:DH * (j + 1)]
                kj = k_ref[pl.ds(HQ_LOC * o + j, 1), b].reshape(SKV, DH)
                vj = v_ref[pl.ds(HQ_LOC * o + j, 1), b].reshape(SKV, DH)
                s_ = lax.dot_general(
                    qj.astype(jnp.bfloat16), kj, (((1,), (1,)), ((), ())),
                    preferred_element_type=jnp.float32) * 0.125
                s_ = jnp.where(mask, s_, -1e9)
                m = jnp.max(s_, axis=1, keepdims=True)
                w = jnp.exp(s_ - m)
                w = w / jnp.sum(w, axis=1, keepdims=True)
                ctx_parts.append(jnp.dot(
                    w.astype(jnp.bfloat16), vj,
                    preferred_element_type=jnp.float32))
            ctx = jnp.concatenate(ctx_parts, axis=1)
            acc[b] = acc[b] + jnp.dot(
                ctx.astype(jnp.bfloat16), wo_blk,
                preferred_element_type=jnp.float32)

    def make_hop(h, rightward):
        if rightward:
            src, dst, tgt = (0 if h == 0 else N_DEV - h), N_DEV - 1 - h, right
            sems = (rq_send, rq_recv, ro_send, ro_recv)
        else:
            src, dst, tgt = (0 if h == 0 else h), h + 1, left
            sems = (lq_send, lq_recv, lo_send, lo_recv)
        rq = pltpu.make_async_remote_copy(
            src_ref=wq_g.at[src], dst_ref=wq_g.at[dst],
            send_sem=sems[0].at[h], recv_sem=sems[1].at[h],
            device_id=(tgt,), device_id_type=pl.DeviceIdType.MESH,
        )
        ro = pltpu.make_async_remote_copy(
            src_ref=wo_g.at[src], dst_ref=wo_g.at[dst],
            send_sem=sems[2].at[h], recv_sem=sems[3].at[h],
            device_id=(tgt,), device_id_type=pl.DeviceIdType.MESH,
        )
        return rq, ro

    hops_r = [make_hop(h, True) for h in range(R_HOPS)]
    hops_l = [make_hop(h, False) for h in range(L_HOPS)]

    for r in hops_r[0] + hops_l[0]:
        r.start()
    contrib(0)

    for h in range(R_HOPS):
        for r in hops_r[h]:
            r.wait_recv()
        if h < L_HOPS:
            for r in hops_l[h]:
                r.wait_recv()
        if h + 1 < R_HOPS:
            for r in hops_r[h + 1]:
                r.start()
        if h + 1 < L_HOPS:
            for r in hops_l[h + 1]:
                r.start()
        for r in hops_r[h]:
            r.wait_send()
        if h < L_HOPS:
            for r in hops_l[h]:
                r.wait_send()
        contrib(N_DEV - 1 - h)
        if h < L_HOPS:
            contrib(h + 1)

    for b in range(B_LOC):
        out_ref[b] = acc[b]

    @functools.partial(pl.run_scoped, second_barrier=pltpu.SemaphoreType.REGULAR)
    def _(second_barrier):
        for nbr in (left, right):
            pl.semaphore_signal(
                second_barrier, inc=1,
                device_id=(nbr,), device_id_type=pl.DeviceIdType.MESH,
            )
        pl.semaphore_wait(second_barrier, 2)


def kernel(x, Wq, K_ext, V_ext, Wo):
    my = lax.axis_index("i")
    k_loc = lax.dynamic_slice_in_dim(K_ext, my * B_LOC, B_LOC, axis=0)
    v_loc = lax.dynamic_slice_in_dim(V_ext, my * B_LOC, B_LOC, axis=0)
    k_t = k_loc.transpose(2, 0, 1, 3).astype(jnp.bfloat16)
    v_t = v_loc.transpose(2, 0, 1, 3).astype(jnp.bfloat16)

    return pl.pallas_call(
        _body,
        out_shape=jax.ShapeDtypeStruct((B_LOC, SQ, D_MODEL), jnp.float32),
        in_specs=[pl.BlockSpec(memory_space=pltpu.VMEM)] * 5,
        out_specs=pl.BlockSpec(memory_space=pltpu.VMEM),
        scratch_shapes=[
            pltpu.VMEM((N_DEV, D_MODEL, D_HEADS_LOC), jnp.bfloat16),
            pltpu.VMEM((N_DEV, D_HEADS_LOC, D_MODEL), jnp.bfloat16),
            pltpu.SemaphoreType.DMA((R_HOPS,)),
            pltpu.SemaphoreType.DMA((R_HOPS,)),
            pltpu.SemaphoreType.DMA((R_HOPS,)),
            pltpu.SemaphoreType.DMA((R_HOPS,)),
            pltpu.SemaphoreType.DMA((L_HOPS,)),
            pltpu.SemaphoreType.DMA((L_HOPS,)),
            pltpu.SemaphoreType.DMA((L_HOPS,)),
            pltpu.SemaphoreType.DMA((L_HOPS,)),
        ],
        compiler_params=pltpu.CompilerParams(
            collective_id=0, vmem_limit_bytes=100 * 1024 * 1024
        ),
    )(x.astype(jnp.bfloat16), Wq.astype(jnp.bfloat16),
      Wo.astype(jnp.bfloat16), k_t, v_t)


# device time: 107636 ns/iter; 2.5807x vs baseline; 1.0075x over previous
import functools

import jax
import jax.numpy as jnp
from jax import lax
from jax.experimental import pallas as pl
from jax.experimental.pallas import tpu as pltpu

N_DEV = 16
B_LOC = 2
SQ = 256
SKV = 256
D_MODEL = 512
HQ_LOC = 4
DH = 64
D_HEADS_LOC = HQ_LOC * DH
BLK = 64
R_HOPS = 8
L_HOPS = 7


def _body(x_ref, wq_ref, wo_ref, k_ref, v_ref, out_ref,
          wq_g, wo_g,
          rq_send, rq_recv, ro_send, ro_recv,
          lq_send, lq_recv, lo_send, lo_recv):
    my = lax.axis_index("i")
    left = lax.rem(my + N_DEV - 1, N_DEV)
    right = lax.rem(my + 1, N_DEV)

    barrier_sem = pltpu.get_barrier_semaphore()
    for nbr in (left, right):
        pl.semaphore_signal(
            barrier_sem, inc=1,
            device_id=(nbr,), device_id_type=pl.DeviceIdType.MESH,
        )
    pl.semaphore_wait(barrier_sem, 2)

    wq_g[0] = wq_ref[:, :]
    wo_g[0] = wo_ref[:, :]

    mask = (lax.broadcasted_iota(jnp.int32, (SQ, SKV), 1) // BLK) <= (
        lax.broadcasted_iota(jnp.int32, (SQ, SKV), 0) // BLK)

    x_all = x_ref[:, :, :].reshape(B_LOC * SQ, D_MODEL)
    acc = [jnp.zeros((B_LOC * SQ, D_MODEL), jnp.float32)]

    def contrib(slot):
        o = lax.rem(my + slot, N_DEV)
        wq_blk = wq_g[slot]
        wo_blk = wo_g[slot]
        q_all = jnp.dot(x_all, wq_blk,
                        preferred_element_type=jnp.float32)
        ctx_rows = []
        for b in range(B_LOC):
            ctx_parts = []
            for j in range(HQ_LOC):
                qj = q_all[SQ * b:SQ * (b + 1), DH * j:DH * (j + 1)]
                kj = k_ref[pl.ds(HQ_LOC * o + j, 1), b].reshape(SKV, DH)
                vj = v_ref[pl.ds(HQ_LOC * o + j, 1), b].reshape(SKV, DH)
                s_ = lax.dot_general(
                    qj.astype(jnp.bfloat16), kj, (((1,), (1,)), ((), ())),
                    preferred_element_type=jnp.float32) * 0.125
                w = jnp.exp(jnp.where(mask, s_, -1e9))
                wsum = jnp.sum(w, axis=1, keepdims=True)
                pv = jnp.dot(w.astype(jnp.bfloat16), vj,
                             preferred_element_type=jnp.float32)
                ctx_parts.append(pv / wsum)
            ctx_rows.append(jnp.concatenate(ctx_parts, axis=1))
        ctx_all = jnp.concatenate(ctx_rows, axis=0)
        acc[0] = acc[0] + jnp.dot(
            ctx_all.astype(jnp.bfloat16), wo_blk,
            preferred_element_type=jnp.float32)

    def make_hop(h, rightward):
        if rightward:
            src, dst, tgt = (0 if h == 0 else N_DEV - h), N_DEV - 1 - h, right
            sems = (rq_send, rq_recv, ro_send, ro_recv)
        else:
            src, dst, tgt = (0 if h == 0 else h), h + 1, left
            sems = (lq_send, lq_recv, lo_send, lo_recv)
        rq = pltpu.make_async_remote_copy(
            src_ref=wq_g.at[src], dst_ref=wq_g.at[dst],
            send_sem=sems[0].at[h], recv_sem=sems[1].at[h],
            device_id=(tgt,), device_id_type=pl.DeviceIdType.MESH,
        )
        ro = pltpu.make_async_remote_copy(
            src_ref=wo_g.at[src], dst_ref=wo_g.at[dst],
            send_sem=sems[2].at[h], recv_sem=sems[3].at[h],
            device_id=(tgt,), device_id_type=pl.DeviceIdType.MESH,
        )
        return rq, ro

    hops_r = [make_hop(h, True) for h in range(R_HOPS)]
    hops_l = [make_hop(h, False) for h in range(L_HOPS)]

    for r in hops_r[0] + hops_l[0]:
        r.start()
    contrib(0)

    for h in range(R_HOPS):
        for r in hops_r[h]:
            r.wait_recv()
        if h < L_HOPS:
            for r in hops_l[h]:
                r.wait_recv()
        if h + 1 < R_HOPS:
            for r in hops_r[h + 1]:
                r.start()
        if h + 1 < L_HOPS:
            for r in hops_l[h + 1]:
                r.start()
        for r in hops_r[h]:
            r.wait_send()
        if h < L_HOPS:
            for r in hops_l[h]:
                r.wait_send()
        contrib(N_DEV - 1 - h)
        if h < L_HOPS:
            contrib(h + 1)

    out_ref[:, :, :] = acc[0].reshape(B_LOC, SQ, D_MODEL)

    @functools.partial(pl.run_scoped, second_barrier=pltpu.SemaphoreType.REGULAR)
    def _(second_barrier):
        for nbr in (left, right):
            pl.semaphore_signal(
                second_barrier, inc=1,
                device_id=(nbr,), device_id_type=pl.DeviceIdType.MESH,
            )
        pl.semaphore_wait(second_barrier, 2)


def kernel(x, Wq, K_ext, V_ext, Wo):
    my = lax.axis_index("i")
    k_loc = lax.dynamic_slice_in_dim(K_ext, my * B_LOC, B_LOC, axis=0)
    v_loc = lax.dynamic_slice_in_dim(V_ext, my * B_LOC, B_LOC, axis=0)
    k_t = k_loc.transpose(2, 0, 1, 3).astype(jnp.bfloat16)
    v_t = v_loc.transpose(2, 0, 1, 3).astype(jnp.bfloat16)

    return pl.pallas_call(
        _body,
        out_shape=jax.ShapeDtypeStruct((B_LOC, SQ, D_MODEL), jnp.float32),
        in_specs=[pl.BlockSpec(memory_space=pltpu.VMEM)] * 5,
        out_specs=pl.BlockSpec(memory_space=pltpu.VMEM),
        scratch_shapes=[
            pltpu.VMEM((N_DEV, D_MODEL, D_HEADS_LOC), jnp.bfloat16),
            pltpu.VMEM((N_DEV, D_HEADS_LOC, D_MODEL), jnp.bfloat16),
            pltpu.SemaphoreType.DMA((R_HOPS,)),
            pltpu.SemaphoreType.DMA((R_HOPS,)),
            pltpu.SemaphoreType.DMA((R_HOPS,)),
            pltpu.SemaphoreType.DMA((R_HOPS,)),
            pltpu.SemaphoreType.DMA((L_HOPS,)),
            pltpu.SemaphoreType.DMA((L_HOPS,)),
            pltpu.SemaphoreType.DMA((L_HOPS,)),
            pltpu.SemaphoreType.DMA((L_HOPS,)),
        ],
        compiler_params=pltpu.CompilerParams(
            collective_id=0, vmem_limit_bytes=100 * 1024 * 1024
        ),
    )(x.astype(jnp.bfloat16), Wq.astype(jnp.bfloat16),
      Wo.astype(jnp.bfloat16), k_t, v_t)
